# baseline (speedup 1.0000x reference)
import jax
import jax.numpy as jnp
from jax import lax
from jax.experimental import pallas as pl

N_VOCAB = 2048
BATCH = 4
SEQ = 8192
SEQ_OUT = 8193
POS_BLK = 128
N_BLKS = 65


def _onehot_block(prev_ref, cur_ref, out_ref):
    j = pl.program_id(0)

    @pl.when(j < N_BLKS - 1)
    def _full():
        prev_last = prev_ref[:, POS_BLK - 1:]
        cur_head = cur_ref[:, : POS_BLK - 1]
        ids = jnp.concatenate([prev_last, cur_head], axis=1)
        lane = lax.broadcasted_iota(jnp.int32, (BATCH, POS_BLK), 1)
        ids = jnp.where((j == 0) & (lane == 0), 0, ids)
        cls = lax.broadcasted_iota(jnp.int32, (POS_BLK, BATCH, N_VOCAB), 2)
        out_ref[...] = (ids.T[:, :, None] == cls).astype(jnp.float32)

    @pl.when(j == N_BLKS - 1)
    def _tail():
        prev_last = prev_ref[:, POS_BLK - 1:]
        cur_head = cur_ref[:, :7]
        ids8 = jnp.concatenate([prev_last, cur_head], axis=1)
        cls8 = lax.broadcasted_iota(jnp.int32, (8, BATCH, N_VOCAB), 2)
        out_ref[pl.ds(0, 8)] = (ids8.T[:, :, None] == cls8).astype(jnp.float32)


def kernel(input_ids):
    ids = input_ids.astype(jnp.int32)
    out2 = pl.pallas_call(
        _onehot_block,
        grid=(N_BLKS,),
        in_specs=[
            pl.BlockSpec((BATCH, POS_BLK), lambda j: (0, jnp.maximum(j - 1, 0))),
            pl.BlockSpec((BATCH, POS_BLK), lambda j: (0, jnp.minimum(j, SEQ // POS_BLK - 1))),
        ],
        out_specs=pl.BlockSpec((POS_BLK, BATCH, N_VOCAB), lambda j: (j, 0, 0)),
        out_shape=jax.ShapeDtypeStruct((SEQ_OUT, BATCH, N_VOCAB), jnp.float32),
    )(ids, ids)
    return jnp.transpose(out2, (1, 0, 2))


# speedup vs baseline: 1.1214x; 1.1214x over previous
import jax
import jax.numpy as jnp
from jax import lax
from jax.experimental import pallas as pl

N_VOCAB = 2048
BATCH = 4
SEQ = 8192
SEQ_OUT = 8193
POS_BLK = 256
N_BLKS = 33


def _onehot_block(prev_ref, cur_ref, out_ref):
    j = pl.program_id(0)

    @pl.when(j < N_BLKS - 1)
    def _full():
        prev_last = prev_ref[:, POS_BLK - 1:]
        cur_head = cur_ref[:, : POS_BLK - 1]
        ids = jnp.concatenate([prev_last, cur_head], axis=1)
        lane = lax.broadcasted_iota(jnp.int32, (BATCH, POS_BLK), 1)
        ids = jnp.where((j == 0) & (lane == 0), 0, ids)
        cls = lax.broadcasted_iota(jnp.int32, (POS_BLK, BATCH, N_VOCAB), 2)
        out_ref[...] = (ids.T[:, :, None] == cls).astype(jnp.float32)

    @pl.when(j == N_BLKS - 1)
    def _tail():
        prev_last = prev_ref[:, POS_BLK - 1:]
        cur_head = cur_ref[:, :7]
        ids8 = jnp.concatenate([prev_last, cur_head], axis=1)
        cls8 = lax.broadcasted_iota(jnp.int32, (8, BATCH, N_VOCAB), 2)
        out_ref[pl.ds(0, 8)] = (ids8.T[:, :, None] == cls8).astype(jnp.float32)


def kernel(input_ids):
    ids = input_ids.astype(jnp.int32)
    out2 = pl.pallas_call(
        _onehot_block,
        grid=(N_BLKS,),
        in_specs=[
            pl.BlockSpec((BATCH, POS_BLK), lambda j: (0, jnp.maximum(j - 1, 0))),
            pl.BlockSpec((BATCH, POS_BLK), lambda j: (0, jnp.minimum(j, SEQ // POS_BLK - 1))),
        ],
        out_specs=pl.BlockSpec((POS_BLK, BATCH, N_VOCAB), lambda j: (j, 0, 0)),
        out_shape=jax.ShapeDtypeStruct((SEQ_OUT, BATCH, N_VOCAB), jnp.float32),
    )(ids, ids)
    return jnp.transpose(out2, (1, 0, 2))
